# baseline (speedup 1.0000x reference)
import functools

import jax
import jax.numpy as jnp
from jax import lax
from jax.experimental import pallas as pl
from jax.experimental.pallas import tpu as pltpu
from jax.experimental.pallas import tpu_sc as plsc

_B_OUT = 4096


@functools.lru_cache(maxsize=None)
def _build(t_rows: int, dim: int):
    lanes = 16
    mesh = plsc.VectorSubcoreMesh(core_axis_name="c", subcore_axis_name="s")
    nc, ns = mesh.num_cores, mesh.num_subcores
    nw = nc * ns
    row_words = t_rows * dim
    reps = 8
    assert _B_OUT % (nw * reps) == 0
    n_chunks = _B_OUT // (nw * reps)
    chunk_words = reps * row_words

    @functools.partial(
        pl.kernel,
        mesh=mesh,
        out_type=jax.ShapeDtypeStruct((_B_OUT // reps, chunk_words), jnp.float32),
        scratch_types=[
            pltpu.VMEM((row_words,), jnp.float32),
            pltpu.VMEM((chunk_words,), jnp.float32),
            pltpu.VMEM((dim,), jnp.float32),
            pltpu.SemaphoreType.DMA,
        ],
    )
    def k(temb_hbm, bias_hbm, out_hbm, tmp_v, big_v, bias_v, sem):
        wid = lax.axis_index("s") * nc + lax.axis_index("c")
        pltpu.sync_copy(temb_hbm, tmp_v)
        pltpu.sync_copy(bias_hbm, bias_v)

        def add_row(j, carry):
            for kk in range(dim // lanes):
                off = j * dim + kk * lanes
                v = tmp_v[pl.ds(off, lanes)] + bias_v[pl.ds(kk * lanes, lanes)]
                for rr in range(reps):
                    big_v[pl.ds(rr * row_words + off, lanes)] = v
            return carry

        lax.fori_loop(0, t_rows, add_row, 0)

        base = wid * n_chunks

        def issue(r, carry):
            pltpu.async_copy(big_v, out_hbm.at[base + r], sem)
            return carry

        lax.fori_loop(0, n_chunks, issue, 0)

        def drain(r, carry):
            pltpu.make_async_copy(big_v, out_hbm.at[base + r], sem).wait()
            return carry

        lax.fori_loop(0, n_chunks, drain, 0)

    return k


def kernel(B, T, temb_weight, type_bias):
    t_rows, dim = temb_weight.shape
    temb_flat = temb_weight.reshape(t_rows * dim)
    bias_flat = type_bias.reshape(dim)
    out = _build(t_rows, dim)(temb_flat, bias_flat)
    return out.reshape(_B_OUT, t_rows, 1, dim)


# speedup vs baseline: 8.5838x; 8.5838x over previous
import functools

import jax
import jax.numpy as jnp
from jax import lax
from jax.experimental import pallas as pl
from jax.experimental.pallas import tpu as pltpu
from jax.experimental.pallas import tpu_sc as plsc

_B_OUT = 4096
_LANES = 16
_SUB = 8
_LANE_T = 128


@functools.lru_cache(maxsize=None)
def _build(t_rows: int, dim: int):
    mesh = plsc.VectorSubcoreMesh(core_axis_name="c", subcore_axis_name="s")
    nc, ns = mesh.num_cores, mesh.num_subcores
    nw = nc * ns
    n_bblk = _B_OUT // _LANE_T
    n_dblk = dim // _SUB
    n_units = t_rows * n_dblk
    pat_words = _SUB * _LANE_T
    unit_words = n_bblk * pat_words
    assert n_units % nw == 0
    units_per_w = n_units // nw
    assert units_per_w % 2 == 0

    vals_per_w = units_per_w * _SUB

    @functools.partial(
        pl.kernel,
        mesh=mesh,
        out_type=jax.ShapeDtypeStruct(
            (t_rows, n_dblk, n_bblk, _SUB, _LANE_T), jnp.float32
        ),
        scratch_types=[
            pltpu.VMEM((vals_per_w,), jnp.float32),
            pltpu.VMEM((dim,), jnp.float32),
            pltpu.VMEM((n_bblk, _SUB, _LANE_T), jnp.float32),
            pltpu.VMEM((n_bblk, _SUB, _LANE_T), jnp.float32),
            pltpu.SemaphoreType.DMA,
            pltpu.SemaphoreType.DMA,
        ],
    )
    def k(temb_hbm, bias_hbm, out_hbm, tab_v, bias_v,
          buf_a, buf_b, sem_a, sem_b):
        wid = lax.axis_index("s") * nc + lax.axis_index("c")
        u0 = wid * units_per_w
        v0 = u0 * _SUB
        pltpu.sync_copy(temb_hbm.at[pl.ds(v0, vals_per_w)], tab_v)
        pltpu.sync_copy(bias_hbm, bias_v)

        def build_unit(vals, buf):
            def rep_body(rep, carry):
                for r in range(_SUB):
                    for j in range(_LANE_T // _LANES):
                        buf[rep, r, pl.ds(j * _LANES, _LANES)] = vals[r]
                return carry

            lax.fori_loop(0, n_bblk, rep_body, 0)

        def pair(p, carry):
            vec = tab_v[pl.ds(_LANES * p, _LANES)]
            bvec = bias_v[pl.ds((v0 + _LANES * p) % dim, _LANES)]
            sv = vec + bvec
            for b, buf, sem in ((0, buf_a, sem_a), (1, buf_b, sem_b)):
                u = u0 + 2 * p + b

                @pl.when(p > 0)
                def _wait():
                    up = u - 2
                    pltpu.make_async_copy(
                        buf, out_hbm.at[up // n_dblk, up % n_dblk], sem
                    ).wait()

                vals = [
                    jnp.full((_LANES,), sv[_SUB * b + r], jnp.float32)
                    for r in range(_SUB)
                ]
                build_unit(vals, buf)
                pltpu.async_copy(buf, out_hbm.at[u // n_dblk, u % n_dblk], sem)
            return carry

        lax.fori_loop(0, units_per_w // 2, pair, 0)
        ua = u0 + units_per_w - 2
        ub = u0 + units_per_w - 1
        pltpu.make_async_copy(
            buf_a, out_hbm.at[ua // n_dblk, ua % n_dblk], sem_a
        ).wait()
        pltpu.make_async_copy(
            buf_b, out_hbm.at[ub // n_dblk, ub % n_dblk], sem_b
        ).wait()

    return k


def kernel(B, T, temb_weight, type_bias):
    t_rows, dim = temb_weight.shape
    temb_flat = temb_weight.reshape(t_rows * dim)
    bias_flat = type_bias.reshape(dim)
    out = _build(t_rows, dim)(temb_flat, bias_flat)
    out = out.transpose(2, 4, 0, 1, 3).reshape(_B_OUT, t_rows, dim)
    return out[:, :, None, :]
